# Initial kernel scaffold; baseline (speedup 1.0000x reference)
import jax
import jax.numpy as jnp
from jax.experimental import pallas as pl


def kernel(x, edge_index, W1, b1, W2, b2, fc_w, fc_b):
    raise NotImplementedError("write your pallas kernel here")


# speedup vs baseline: 11.8325x; 11.8325x over previous
import functools

import jax
import jax.numpy as jnp
from jax import lax
from jax.experimental import pallas as pl
from jax.experimental.pallas import tpu as pltpu
from jax.experimental.pallas import tpu_sc as plsc

NC = 2
NS = 16
NW = NC * NS
LANE = 128

_MESH = plsc.VectorSubcoreMesh(
    core_axis_name="c", subcore_axis_name="s", num_cores=NC, num_subcores=NS
)
_SC_PARAMS = pltpu.CompilerParams(use_tc_tiling_on_sc=False)


def _edge_prep(n):
    def body(ei_ref, out_ref):
        s = ei_ref[0]
        d = ei_ref[1]
        m = s != d
        out_ref[0] = s
        out_ref[1] = jnp.where(m, d, n)
        out_ref[2] = jnp.where(m, s, n)
    return body


def _make_deg_kernel(n_pad, rpt, rps):
    @functools.partial(
        pl.kernel,
        out_type=jax.ShapeDtypeStruct((NC, n_pad, 16), jnp.float32),
        mesh=_MESH,
        compiler_params=_SC_PARAMS,
        scratch_types=[
            pltpu.VMEM((rpt, LANE), jnp.int32),
            pltpu.VMEM((LANE, 16), jnp.float32),
            pltpu.VMEM_SHARED((n_pad, 16), jnp.float32),
        ],
    )
    def deg_kernel(idx_hbm, ones_hbm, zeros_hbm, out_hbm, idx_v, ones_v, acc):
        c = lax.axis_index("c")
        s = lax.axis_index("s")
        wid = c * NS + s
        pltpu.sync_copy(zeros_hbm, acc.at[pl.ds(s * rps, rps)])
        pltpu.sync_copy(ones_hbm, ones_v)
        pltpu.sync_copy(idx_hbm.at[pl.ds(wid * rpt, rpt)], idx_v)
        plsc.subcore_barrier()

        def body(j, carry):
            pltpu.sync_copy(ones_v, acc.at[idx_v.at[j]], add=True)
            return carry

        lax.fori_loop(0, rpt, body, 0)
        plsc.subcore_barrier()
        pltpu.sync_copy(acc.at[pl.ds(s * rps, rps)],
                        out_hbm.at[c, pl.ds(s * rps, rps)])

    return deg_kernel


def _make_prop_kernel(n_pad, rpt, rps, dh):
    @functools.partial(
        pl.kernel,
        out_type=jax.ShapeDtypeStruct((NC, n_pad, dh), jnp.float32),
        mesh=_MESH,
        compiler_params=_SC_PARAMS,
        scratch_types=[
            pltpu.VMEM((rpt, LANE), jnp.int32),
            pltpu.VMEM((rpt, LANE), jnp.int32),
            pltpu.VMEM((LANE, dh), jnp.float32),
            pltpu.VMEM_SHARED((n_pad, dh), jnp.float32),
        ],
    )
    def prop_kernel(table_hbm, gidx_hbm, sidx_hbm, zeros_hbm, out_hbm,
                    gi_v, si_v, rows_v, acc):
        c = lax.axis_index("c")
        s = lax.axis_index("s")
        wid = c * NS + s
        pltpu.sync_copy(zeros_hbm, acc.at[pl.ds(s * rps, rps)])
        pltpu.sync_copy(gidx_hbm.at[pl.ds(wid * rpt, rpt)], gi_v)
        pltpu.sync_copy(sidx_hbm.at[pl.ds(wid * rpt, rpt)], si_v)
        plsc.subcore_barrier()

        def body(j, carry):
            pltpu.sync_copy(table_hbm.at[gi_v.at[j]], rows_v)
            pltpu.sync_copy(rows_v, acc.at[si_v.at[j]], add=True)
            return carry

        lax.fori_loop(0, rpt, body, 0)
        plsc.subcore_barrier()
        pltpu.sync_copy(acc.at[pl.ds(s * rps, rps)],
                        out_hbm.at[c, pl.ds(s * rps, rps)])

    return prop_kernel


def _stage_first(x_ref, wc_ref, d0_ref, d1_ref, o_ref, dinv_ref):
    deg = d0_ref[...] + d1_ref[...]
    dinv = jnp.where(deg > 0.0, lax.rsqrt(deg), 0.0)
    y = jnp.dot(x_ref[...], wc_ref[...], preferred_element_type=jnp.float32)
    col = lax.broadcasted_iota(jnp.int32, (1, y.shape[1]), 1)
    o_ref[...] = y * jnp.where(col >= 64, dinv, 1.0)
    dinv_ref[...] = dinv


def _stage_mid(y1_ref, s0_ref, s1_ref, dinv_ref, o_ref):
    dinv = dinv_ref[...]
    o_ref[...] = dinv * (y1_ref[...] - 2.0 * dinv * (s0_ref[...] + s1_ref[...]))


def _stage_layer(y0_ref, t0_ref, t1_ref, dinv_ref, b_ref, wc_ref,
                 o_ref, dinv_keep_ref):
    dinv = dinv_ref[...]
    h = jax.nn.relu(y0_ref[...] - dinv * (t0_ref[...] + t1_ref[...]) + b_ref[...])
    z = jnp.dot(h, wc_ref[...], preferred_element_type=jnp.float32)
    col = lax.broadcasted_iota(jnp.int32, (1, z.shape[1]), 1)
    o_ref[...] = z * jnp.where(col >= 64, dinv, 1.0)
    dinv_keep_ref[...] = dinv


def _stage_final(z0_ref, v0_ref, v1_ref, dinv_ref, b_ref, fw_ref, fb_ref, o_ref):
    dinv = dinv_ref[...]
    h = jax.nn.relu(z0_ref[...] - dinv * (v0_ref[...] + v1_ref[...]) + b_ref[...])
    o_ref[...] = (
        jnp.dot(h, fw_ref[...], preferred_element_type=jnp.float32) + fb_ref[...]
    )


def kernel(x, edge_index, W1, b1, W2, b2, fc_w, fc_b):
    n, d_in = x.shape
    dh = W1.shape[2]
    e = edge_index.shape[1]

    epb = NW * 8 * LANE
    e_pad = -(-e // epb) * epb
    rows = e_pad // LANE
    rpt = rows // NW
    ei = jnp.pad(edge_index, ((0, 0), (0, e_pad - e)))
    ei = ei.reshape(2, rows, LANE)

    n_pad = -(-(n + 1) // (NS * 8)) * (NS * 8)
    rps = n_pad // NS

    idx3 = pl.pallas_call(
        _edge_prep(n),
        out_shape=jax.ShapeDtypeStruct((3, rows, LANE), jnp.int32),
    )(ei)
    src_g = idx3[0]
    dst_r = idx3[1]
    src_r = idx3[2]

    zeros16 = jnp.zeros((rps, 16), jnp.float32)
    ones16 = jnp.ones((LANE, 16), jnp.float32)
    zeros_dh = jnp.zeros((rps, dh), jnp.float32)

    deg_kernel = _make_deg_kernel(n_pad, rpt, rps)
    degp = deg_kernel(src_r, ones16, zeros16)
    d0 = degp[0, :n, 0:1]
    d1 = degp[1, :n, 0:1]

    prop_kernel = _make_prop_kernel(n_pad, rpt, rps, dh)

    def run_prop(table):
        p = prop_kernel(table, src_g, dst_r, zeros_dh)
        return p[0, :n, :], p[1, :n, :]

    wc1 = jnp.concatenate([W1[0] - W1[2], W1[1], W1[2]], axis=1)
    ym, dinv = pl.pallas_call(
        _stage_first,
        out_shape=(
            jax.ShapeDtypeStruct((n, 3 * dh), jnp.float32),
            jax.ShapeDtypeStruct((n, 1), jnp.float32),
        ),
    )(x, wc1, d0, d1)

    s0, s1 = run_prop(ym[:, 2 * dh:])
    gc = pl.pallas_call(
        _stage_mid,
        out_shape=jax.ShapeDtypeStruct((n, dh), jnp.float32),
    )(ym[:, dh:2 * dh], s0, s1, dinv)

    t0, t1 = run_prop(gc)

    wc2 = jnp.concatenate([W2[0] - W2[2], W2[1], W2[2]], axis=1)
    zm, dinv2 = pl.pallas_call(
        _stage_layer,
        out_shape=(
            jax.ShapeDtypeStruct((n, 3 * dh), jnp.float32),
            jax.ShapeDtypeStruct((n, 1), jnp.float32),
        ),
    )(ym[:, :dh], t0, t1, dinv, b1.reshape(1, dh), wc2)

    u0, u1 = run_prop(zm[:, 2 * dh:])
    gc2 = pl.pallas_call(
        _stage_mid,
        out_shape=jax.ShapeDtypeStruct((n, dh), jnp.float32),
    )(zm[:, dh:2 * dh], u0, u1, dinv2)

    v0, v1 = run_prop(gc2)

    out = pl.pallas_call(
        _stage_final,
        out_shape=jax.ShapeDtypeStruct((n, 1), jnp.float32),
    )(zm[:, :dh], v0, v1, dinv2, b2.reshape(1, dh), fc_w, fc_b.reshape(1, 1))
    return out


# speedup vs baseline: 14.8749x; 1.2571x over previous
import functools

import jax
import jax.numpy as jnp
from jax import lax
from jax.experimental import pallas as pl
from jax.experimental.pallas import tpu as pltpu
from jax.experimental.pallas import tpu_sc as plsc

NC = 2
NS = 16
NW = NC * NS
LANE = 128

_MESH = plsc.VectorSubcoreMesh(
    core_axis_name="c", subcore_axis_name="s", num_cores=NC, num_subcores=NS
)
_SC_PARAMS = pltpu.CompilerParams(use_tc_tiling_on_sc=False)


def _edge_prep(n):
    def body(ei_ref, out_ref):
        s = ei_ref[0]
        d = ei_ref[1]
        m = s != d
        out_ref[0] = s
        out_ref[1] = jnp.where(m, d, n)
        out_ref[2] = jnp.where(m, s, n)
    return body


def _make_deg_kernel(n_pad, rpt, rps):
    @functools.partial(
        pl.kernel,
        out_type=jax.ShapeDtypeStruct((NC, n_pad, 16), jnp.float32),
        mesh=_MESH,
        compiler_params=_SC_PARAMS,
        scratch_types=[
            pltpu.VMEM((rpt, LANE), jnp.int32),
            pltpu.VMEM((LANE, 16), jnp.float32),
            pltpu.VMEM_SHARED((n_pad, 16), jnp.float32),
        ],
    )
    def deg_kernel(idx_hbm, ones_hbm, zeros_hbm, out_hbm, idx_v, ones_v, acc):
        c = lax.axis_index("c")
        s = lax.axis_index("s")
        wid = c * NS + s
        pltpu.sync_copy(zeros_hbm, acc.at[pl.ds(s * rps, rps)])
        pltpu.sync_copy(ones_hbm, ones_v)
        pltpu.sync_copy(idx_hbm.at[pl.ds(wid * rpt, rpt)], idx_v)
        plsc.subcore_barrier()

        def body(j, carry):
            pltpu.sync_copy(ones_v, acc.at[idx_v.at[j]], add=True)
            return carry

        lax.fori_loop(0, rpt, body, 0)
        plsc.subcore_barrier()
        pltpu.sync_copy(acc.at[pl.ds(s * rps, rps)],
                        out_hbm.at[c, pl.ds(s * rps, rps)])

    return deg_kernel


def _make_prop_kernel(n_pad, rpt, rps, dh):
    nbuf = min(8, rpt)
    groups = rpt // nbuf
    rem = rpt - groups * nbuf

    @functools.partial(
        pl.kernel,
        out_type=jax.ShapeDtypeStruct((NC, n_pad, dh), jnp.float32),
        mesh=_MESH,
        compiler_params=_SC_PARAMS,
        scratch_types=[
            pltpu.VMEM((rpt, LANE), jnp.int32),
            pltpu.VMEM((rpt, LANE), jnp.int32),
            pltpu.VMEM((nbuf, LANE, dh), jnp.float32),
            pltpu.VMEM_SHARED((n_pad, dh), jnp.float32),
            pltpu.SemaphoreType.DMA((nbuf,)),
        ],
    )
    def prop_kernel(table_hbm, gidx_hbm, sidx_hbm, zeros_hbm, out_hbm,
                    gi_v, si_v, rows_v, acc, sem):
        c = lax.axis_index("c")
        s = lax.axis_index("s")
        wid = c * NS + s
        pltpu.sync_copy(gidx_hbm.at[pl.ds(wid * rpt, rpt)], gi_v)
        pltpu.sync_copy(sidx_hbm.at[pl.ds(wid * rpt, rpt)], si_v)
        pltpu.sync_copy(zeros_hbm, acc.at[pl.ds(s * rps, rps)])
        for b in range(nbuf):
            pltpu.async_copy(table_hbm.at[gi_v.at[b]], rows_v.at[b], sem.at[b])
        plsc.subcore_barrier()

        def group(g, carry):
            j0 = g * nbuf
            for b in range(nbuf):
                pltpu.make_async_copy(
                    table_hbm.at[gi_v.at[0]], rows_v.at[b], sem.at[b]
                ).wait()
                pltpu.sync_copy(rows_v.at[b], acc.at[si_v.at[j0 + b]], add=True)

                @pl.when(j0 + nbuf + b < rpt)
                def _():
                    pltpu.async_copy(
                        table_hbm.at[gi_v.at[j0 + nbuf + b]],
                        rows_v.at[b], sem.at[b],
                    )
            return carry

        lax.fori_loop(0, groups, group, 0)
        for b in range(rem):
            j = groups * nbuf + b
            pltpu.make_async_copy(
                table_hbm.at[gi_v.at[0]], rows_v.at[b], sem.at[b]
            ).wait()
            pltpu.sync_copy(rows_v.at[b], acc.at[si_v.at[j]], add=True)
        plsc.subcore_barrier()
        pltpu.sync_copy(acc.at[pl.ds(s * rps, rps)],
                        out_hbm.at[c, pl.ds(s * rps, rps)])

    return prop_kernel


def _stage_first(x_ref, wc_ref, d0_ref, d1_ref, o_ref, dinv_ref):
    deg = d0_ref[...] + d1_ref[...]
    dinv = jnp.where(deg > 0.0, lax.rsqrt(deg), 0.0)
    y = jnp.dot(x_ref[...], wc_ref[...], preferred_element_type=jnp.float32)
    col = lax.broadcasted_iota(jnp.int32, (1, y.shape[1]), 1)
    o_ref[...] = y * jnp.where(col >= 64, dinv, 1.0)
    dinv_ref[...] = dinv


def _stage_mid(y1_ref, s0_ref, s1_ref, dinv_ref, o_ref):
    dinv = dinv_ref[...]
    o_ref[...] = dinv * (y1_ref[...] - 2.0 * dinv * (s0_ref[...] + s1_ref[...]))


def _stage_layer(y0_ref, t0_ref, t1_ref, dinv_ref, b_ref, wc_ref,
                 o_ref, dinv_keep_ref):
    dinv = dinv_ref[...]
    h = jax.nn.relu(y0_ref[...] - dinv * (t0_ref[...] + t1_ref[...]) + b_ref[...])
    z = jnp.dot(h, wc_ref[...], preferred_element_type=jnp.float32)
    col = lax.broadcasted_iota(jnp.int32, (1, z.shape[1]), 1)
    o_ref[...] = z * jnp.where(col >= 64, dinv, 1.0)
    dinv_keep_ref[...] = dinv


def _stage_final(z0_ref, v0_ref, v1_ref, dinv_ref, b_ref, fw_ref, fb_ref, o_ref):
    dinv = dinv_ref[...]
    h = jax.nn.relu(z0_ref[...] - dinv * (v0_ref[...] + v1_ref[...]) + b_ref[...])
    o_ref[...] = (
        jnp.dot(h, fw_ref[...], preferred_element_type=jnp.float32) + fb_ref[...]
    )


def kernel(x, edge_index, W1, b1, W2, b2, fc_w, fc_b):
    n, d_in = x.shape
    dh = W1.shape[2]
    e = edge_index.shape[1]

    epb = NW * 8 * LANE
    e_pad = -(-e // epb) * epb
    rows = e_pad // LANE
    rpt = rows // NW
    ei = jnp.pad(edge_index, ((0, 0), (0, e_pad - e)))
    ei = ei.reshape(2, rows, LANE)

    n_pad = -(-(n + 1) // (NS * 8)) * (NS * 8)
    rps = n_pad // NS

    idx3 = pl.pallas_call(
        _edge_prep(n),
        out_shape=jax.ShapeDtypeStruct((3, rows, LANE), jnp.int32),
    )(ei)
    src_g = idx3[0]
    dst_r = idx3[1]
    src_r = idx3[2]

    zeros16 = jnp.zeros((rps, 16), jnp.float32)
    ones16 = jnp.ones((LANE, 16), jnp.float32)
    zeros_dh = jnp.zeros((rps, dh), jnp.float32)

    deg_kernel = _make_deg_kernel(n_pad, rpt, rps)
    degp = deg_kernel(src_r, ones16, zeros16)
    d0 = degp[0, :n, 0:1]
    d1 = degp[1, :n, 0:1]

    prop_kernel = _make_prop_kernel(n_pad, rpt, rps, dh)

    def run_prop(table):
        p = prop_kernel(table, src_g, dst_r, zeros_dh)
        return p[0, :n, :], p[1, :n, :]

    wc1 = jnp.concatenate([W1[0] - W1[2], W1[1], W1[2]], axis=1)
    ym, dinv = pl.pallas_call(
        _stage_first,
        out_shape=(
            jax.ShapeDtypeStruct((n, 3 * dh), jnp.float32),
            jax.ShapeDtypeStruct((n, 1), jnp.float32),
        ),
    )(x, wc1, d0, d1)

    s0, s1 = run_prop(ym[:, 2 * dh:])
    gc = pl.pallas_call(
        _stage_mid,
        out_shape=jax.ShapeDtypeStruct((n, dh), jnp.float32),
    )(ym[:, dh:2 * dh], s0, s1, dinv)

    t0, t1 = run_prop(gc)

    wc2 = jnp.concatenate([W2[0] - W2[2], W2[1], W2[2]], axis=1)
    zm, dinv2 = pl.pallas_call(
        _stage_layer,
        out_shape=(
            jax.ShapeDtypeStruct((n, 3 * dh), jnp.float32),
            jax.ShapeDtypeStruct((n, 1), jnp.float32),
        ),
    )(ym[:, :dh], t0, t1, dinv, b1.reshape(1, dh), wc2)

    u0, u1 = run_prop(zm[:, 2 * dh:])
    gc2 = pl.pallas_call(
        _stage_mid,
        out_shape=jax.ShapeDtypeStruct((n, dh), jnp.float32),
    )(zm[:, dh:2 * dh], u0, u1, dinv2)

    v0, v1 = run_prop(gc2)

    out = pl.pallas_call(
        _stage_final,
        out_shape=jax.ShapeDtypeStruct((n, 1), jnp.float32),
    )(zm[:, :dh], v0, v1, dinv2, b2.reshape(1, dh), fc_w, fc_b.reshape(1, 1))
    return out


# speedup vs baseline: 26.1686x; 1.7593x over previous
import functools

import jax
import jax.numpy as jnp
from jax import lax
from jax.experimental import pallas as pl
from jax.experimental.pallas import tpu as pltpu
from jax.experimental.pallas import tpu_sc as plsc

NC = 2
NS = 16
NW = NC * NS
LANE = 128

_MESH = plsc.VectorSubcoreMesh(
    core_axis_name="c", subcore_axis_name="s", num_cores=NC, num_subcores=NS
)
_SC_PARAMS = pltpu.CompilerParams(use_tc_tiling_on_sc=False)


def _edge_prep(n):
    def body(ei_ref, out_ref):
        s = ei_ref[0]
        d = ei_ref[1]
        m = s != d
        out_ref[0] = s
        out_ref[1] = jnp.where(m, d, n)
        out_ref[2] = jnp.where(m, s, n)
    return body


def _make_deg_kernel(n_pad, rpt, rps):
    @functools.partial(
        pl.kernel,
        out_type=jax.ShapeDtypeStruct((NC, n_pad, 16), jnp.float32),
        mesh=_MESH,
        compiler_params=_SC_PARAMS,
        scratch_types=[
            pltpu.VMEM((rpt, LANE), jnp.int32),
            pltpu.VMEM((LANE, 16), jnp.float32),
            pltpu.VMEM_SHARED((n_pad, 16), jnp.float32),
        ],
    )
    def deg_kernel(idx_hbm, ones_hbm, zeros_hbm, out_hbm, idx_v, ones_v, acc):
        c = lax.axis_index("c")
        s = lax.axis_index("s")
        wid = c * NS + s
        pltpu.sync_copy(zeros_hbm, acc.at[pl.ds(s * rps, rps)])
        pltpu.sync_copy(ones_hbm, ones_v)
        pltpu.sync_copy(idx_hbm.at[pl.ds(wid * rpt, rpt)], idx_v)
        plsc.subcore_barrier()

        def body(j, carry):
            pltpu.sync_copy(ones_v, acc.at[idx_v.at[j]], add=True)
            return carry

        lax.fori_loop(0, rpt, body, 0)
        plsc.subcore_barrier()
        pltpu.sync_copy(acc.at[pl.ds(s * rps, rps)],
                        out_hbm.at[c, pl.ds(s * rps, rps)])

    return deg_kernel


def _make_prop_kernel(n_pad, rpt, rps, dh):
    nbuf = min(8, rpt)
    groups = rpt // nbuf
    rem = rpt - groups * nbuf

    @functools.partial(
        pl.kernel,
        out_type=jax.ShapeDtypeStruct((NC, n_pad, dh), jnp.float32),
        mesh=_MESH,
        compiler_params=_SC_PARAMS,
        scratch_types=[
            pltpu.VMEM((rpt, LANE), jnp.int32),
            pltpu.VMEM((rpt, LANE), jnp.int32),
            pltpu.VMEM((nbuf, LANE, dh), jnp.float32),
            pltpu.VMEM_SHARED((n_pad, dh), jnp.float32),
            pltpu.VMEM_SHARED((n_pad, dh), jnp.float32),
            pltpu.SemaphoreType.DMA((nbuf,)),
        ],
    )
    def prop_kernel(table_hbm, gidx_hbm, sidx_hbm, zeros_hbm, out_hbm,
                    gi_v, si_v, rows_v, acc, table_sp, sem):
        c = lax.axis_index("c")
        s = lax.axis_index("s")
        wid = c * NS + s
        pltpu.sync_copy(table_hbm.at[pl.ds(s * rps, rps)],
                        table_sp.at[pl.ds(s * rps, rps)])
        pltpu.sync_copy(gidx_hbm.at[pl.ds(wid * rpt, rpt)], gi_v)
        pltpu.sync_copy(sidx_hbm.at[pl.ds(wid * rpt, rpt)], si_v)
        pltpu.sync_copy(zeros_hbm, acc.at[pl.ds(s * rps, rps)])
        plsc.subcore_barrier()
        for b in range(nbuf):
            pltpu.async_copy(table_sp.at[gi_v.at[b]], rows_v.at[b], sem.at[b])

        def group(g, carry):
            j0 = g * nbuf
            for b in range(nbuf):
                pltpu.make_async_copy(
                    table_sp.at[gi_v.at[0]], rows_v.at[b], sem.at[b]
                ).wait()
                pltpu.sync_copy(rows_v.at[b], acc.at[si_v.at[j0 + b]], add=True)

                @pl.when(j0 + nbuf + b < rpt)
                def _():
                    pltpu.async_copy(
                        table_sp.at[gi_v.at[j0 + nbuf + b]],
                        rows_v.at[b], sem.at[b],
                    )
            return carry

        lax.fori_loop(0, groups, group, 0)
        for b in range(rem):
            j = groups * nbuf + b
            pltpu.make_async_copy(
                table_sp.at[gi_v.at[0]], rows_v.at[b], sem.at[b]
            ).wait()
            pltpu.sync_copy(rows_v.at[b], acc.at[si_v.at[j]], add=True)
        plsc.subcore_barrier()
        pltpu.sync_copy(acc.at[pl.ds(s * rps, rps)],
                        out_hbm.at[c, pl.ds(s * rps, rps)])

    return prop_kernel


def _stage_first(x_ref, wc_ref, d0_ref, d1_ref, o_ref, dinv_ref):
    deg = d0_ref[...] + d1_ref[...]
    dinv = jnp.where(deg > 0.0, lax.rsqrt(deg), 0.0)
    y = jnp.dot(x_ref[...], wc_ref[...], preferred_element_type=jnp.float32)
    col = lax.broadcasted_iota(jnp.int32, (1, y.shape[1]), 1)
    o_ref[...] = y * jnp.where(col >= 64, dinv, 1.0)
    dinv_ref[...] = dinv


def _stage_mid(y1_ref, s0_ref, s1_ref, dinv_ref, o_ref):
    dinv = dinv_ref[...]
    o_ref[...] = dinv * (y1_ref[...] - 2.0 * dinv * (s0_ref[...] + s1_ref[...]))


def _stage_layer(y0_ref, t0_ref, t1_ref, dinv_ref, b_ref, wc_ref,
                 o_ref, dinv_keep_ref):
    dinv = dinv_ref[...]
    h = jax.nn.relu(y0_ref[...] - dinv * (t0_ref[...] + t1_ref[...]) + b_ref[...])
    z = jnp.dot(h, wc_ref[...], preferred_element_type=jnp.float32)
    col = lax.broadcasted_iota(jnp.int32, (1, z.shape[1]), 1)
    o_ref[...] = z * jnp.where(col >= 64, dinv, 1.0)
    dinv_keep_ref[...] = dinv


def _stage_final(z0_ref, v0_ref, v1_ref, dinv_ref, b_ref, fw_ref, fb_ref, o_ref):
    dinv = dinv_ref[...]
    h = jax.nn.relu(z0_ref[...] - dinv * (v0_ref[...] + v1_ref[...]) + b_ref[...])
    o_ref[...] = (
        jnp.dot(h, fw_ref[...], preferred_element_type=jnp.float32) + fb_ref[...]
    )


def kernel(x, edge_index, W1, b1, W2, b2, fc_w, fc_b):
    n, d_in = x.shape
    dh = W1.shape[2]
    e = edge_index.shape[1]

    epb = NW * 8 * LANE
    e_pad = -(-e // epb) * epb
    rows = e_pad // LANE
    rpt = rows // NW
    ei = jnp.pad(edge_index, ((0, 0), (0, e_pad - e)))
    ei = ei.reshape(2, rows, LANE)

    n_pad = -(-(n + 1) // (NS * 8)) * (NS * 8)
    rps = n_pad // NS

    idx3 = pl.pallas_call(
        _edge_prep(n),
        out_shape=jax.ShapeDtypeStruct((3, rows, LANE), jnp.int32),
    )(ei)
    src_g = idx3[0]
    dst_r = idx3[1]
    src_r = idx3[2]

    zeros16 = jnp.zeros((rps, 16), jnp.float32)
    ones16 = jnp.ones((LANE, 16), jnp.float32)
    zeros_dh = jnp.zeros((rps, dh), jnp.float32)

    deg_kernel = _make_deg_kernel(n_pad, rpt, rps)
    degp = deg_kernel(src_r, ones16, zeros16)
    d0 = degp[0, :n, 0:1]
    d1 = degp[1, :n, 0:1]

    prop_kernel = _make_prop_kernel(n_pad, rpt, rps, dh)

    def run_prop(table):
        tpad = jnp.pad(table, ((0, n_pad - n), (0, 0)))
        p = prop_kernel(tpad, src_g, dst_r, zeros_dh)
        return p[0, :n, :], p[1, :n, :]

    wc1 = jnp.concatenate([W1[0] - W1[2], W1[1], W1[2]], axis=1)
    ym, dinv = pl.pallas_call(
        _stage_first,
        out_shape=(
            jax.ShapeDtypeStruct((n, 3 * dh), jnp.float32),
            jax.ShapeDtypeStruct((n, 1), jnp.float32),
        ),
    )(x, wc1, d0, d1)

    s0, s1 = run_prop(ym[:, 2 * dh:])
    gc = pl.pallas_call(
        _stage_mid,
        out_shape=jax.ShapeDtypeStruct((n, dh), jnp.float32),
    )(ym[:, dh:2 * dh], s0, s1, dinv)

    t0, t1 = run_prop(gc)

    wc2 = jnp.concatenate([W2[0] - W2[2], W2[1], W2[2]], axis=1)
    zm, dinv2 = pl.pallas_call(
        _stage_layer,
        out_shape=(
            jax.ShapeDtypeStruct((n, 3 * dh), jnp.float32),
            jax.ShapeDtypeStruct((n, 1), jnp.float32),
        ),
    )(ym[:, :dh], t0, t1, dinv, b1.reshape(1, dh), wc2)

    u0, u1 = run_prop(zm[:, 2 * dh:])
    gc2 = pl.pallas_call(
        _stage_mid,
        out_shape=jax.ShapeDtypeStruct((n, dh), jnp.float32),
    )(zm[:, dh:2 * dh], u0, u1, dinv2)

    v0, v1 = run_prop(gc2)

    out = pl.pallas_call(
        _stage_final,
        out_shape=jax.ShapeDtypeStruct((n, 1), jnp.float32),
    )(zm[:, :dh], v0, v1, dinv2, b2.reshape(1, dh), fc_w, fc_b.reshape(1, 1))
    return out


# speedup vs baseline: 29.2697x; 1.1185x over previous
import functools

import jax
import jax.numpy as jnp
from jax import lax
from jax.experimental import pallas as pl
from jax.experimental.pallas import tpu as pltpu
from jax.experimental.pallas import tpu_sc as plsc

NC = 2
NS = 16
NW = NC * NS
LANE = 128

_MESH = plsc.VectorSubcoreMesh(
    core_axis_name="c", subcore_axis_name="s", num_cores=NC, num_subcores=NS
)
_SC_PARAMS = pltpu.CompilerParams(use_tc_tiling_on_sc=False)


def _edge_prep(n):
    def body(ei_ref, out_ref):
        s = ei_ref[0]
        d = ei_ref[1]
        m = s != d
        out_ref[0] = s
        out_ref[1] = jnp.where(m, d, n)
        out_ref[2] = jnp.where(m, s, n)
    return body


def _make_deg_kernel(n_pad, rpt, rps):
    @functools.partial(
        pl.kernel,
        out_type=jax.ShapeDtypeStruct((NC, n_pad, 16), jnp.float32),
        mesh=_MESH,
        compiler_params=_SC_PARAMS,
        scratch_types=[
            pltpu.VMEM((rpt, LANE), jnp.int32),
            pltpu.VMEM((LANE, 16), jnp.float32),
            pltpu.VMEM_SHARED((n_pad, 16), jnp.float32),
        ],
    )
    def deg_kernel(idx_hbm, ones_hbm, zeros_hbm, out_hbm, idx_v, ones_v, acc):
        c = lax.axis_index("c")
        s = lax.axis_index("s")
        wid = c * NS + s
        pltpu.sync_copy(zeros_hbm, acc.at[pl.ds(s * rps, rps)])
        pltpu.sync_copy(ones_hbm, ones_v)
        pltpu.sync_copy(idx_hbm.at[pl.ds(wid * rpt, rpt)], idx_v)
        plsc.subcore_barrier()

        def body(j, carry):
            pltpu.sync_copy(ones_v, acc.at[idx_v.at[j]], add=True)
            return carry

        lax.fori_loop(0, rpt, body, 0)
        plsc.subcore_barrier()
        pltpu.sync_copy(acc.at[pl.ds(s * rps, rps)],
                        out_hbm.at[c, pl.ds(s * rps, rps)])

    return deg_kernel


def _make_prop_kernel(n_pad, rpt, rps, dh):
    nbuf = min(8, rpt)
    groups = rpt // nbuf
    rem = rpt - groups * nbuf

    @functools.partial(
        pl.kernel,
        out_type=jax.ShapeDtypeStruct((NC, n_pad, dh), jnp.float32),
        mesh=_MESH,
        compiler_params=_SC_PARAMS,
        scratch_types=[
            pltpu.VMEM((rpt, LANE), jnp.int32),
            pltpu.VMEM((rpt, LANE), jnp.int32),
            pltpu.VMEM((nbuf, LANE, dh), jnp.float32),
            pltpu.VMEM_SHARED((n_pad, dh), jnp.float32),
            pltpu.VMEM_SHARED((n_pad, dh), jnp.float32),
            pltpu.SemaphoreType.DMA((nbuf,)),
        ],
    )
    def prop_kernel(table_hbm, gidx_hbm, sidx_hbm, zeros_hbm, out_hbm,
                    gi_v, si_v, rows_v, acc, table_sp, sem):
        c = lax.axis_index("c")
        s = lax.axis_index("s")
        wid = c * NS + s
        pltpu.sync_copy(table_hbm.at[pl.ds(s * rps, rps)],
                        table_sp.at[pl.ds(s * rps, rps)])
        pltpu.sync_copy(gidx_hbm.at[pl.ds(wid * rpt, rpt)], gi_v)
        pltpu.sync_copy(sidx_hbm.at[pl.ds(wid * rpt, rpt)], si_v)
        pltpu.sync_copy(zeros_hbm, acc.at[pl.ds(s * rps, rps)])
        plsc.subcore_barrier()
        for b in range(nbuf):
            pltpu.async_copy(table_sp.at[gi_v.at[b]], rows_v.at[b], sem.at[b])

        def group(g, carry):
            j0 = g * nbuf
            for b in range(nbuf):
                pltpu.make_async_copy(
                    table_sp.at[gi_v.at[0]], rows_v.at[b], sem.at[b]
                ).wait()
                pltpu.sync_copy(rows_v.at[b], acc.at[si_v.at[j0 + b]], add=True)

                @pl.when(j0 + nbuf + b < rpt)
                def _():
                    pltpu.async_copy(
                        table_sp.at[gi_v.at[j0 + nbuf + b]],
                        rows_v.at[b], sem.at[b],
                    )
            return carry

        lax.fori_loop(0, groups, group, 0)
        for b in range(rem):
            j = groups * nbuf + b
            pltpu.make_async_copy(
                table_sp.at[gi_v.at[0]], rows_v.at[b], sem.at[b]
            ).wait()
            pltpu.sync_copy(rows_v.at[b], acc.at[si_v.at[j]], add=True)
        plsc.subcore_barrier()
        pltpu.sync_copy(acc.at[pl.ds(s * rps, rps)],
                        out_hbm.at[c, pl.ds(s * rps, rps)])

    return prop_kernel


def _make_stage_first(n, n_pad, dh):
    def body(x_ref, wc_ref, degp_ref, table_ref, y01_ref, dinv_ref):
        deg = degp_ref[0, :n, 0:1] + degp_ref[1, :n, 0:1]
        dinv = jnp.where(deg > 0.0, lax.rsqrt(deg), 0.0)
        y = jnp.dot(x_ref[...], wc_ref[...], preferred_element_type=jnp.float32)
        y01_ref[...] = y[:, :2 * dh]
        table_ref[...] = jnp.pad(dinv * y[:, 2 * dh:], ((0, n_pad - n), (0, 0)))
        dinv_ref[...] = dinv
    return body


def _make_stage_mid(n, n_pad, dh):
    def body(y01_ref, p_ref, dinv_ref, table_ref):
        dinv = dinv_ref[...]
        ssum = p_ref[0, :n, :] + p_ref[1, :n, :]
        gc = dinv * (y01_ref[:, dh:] - 2.0 * dinv * ssum)
        table_ref[...] = jnp.pad(gc, ((0, n_pad - n), (0, 0)))
    return body


def _make_stage_layer(n, n_pad, dh):
    def body(y01_ref, p_ref, dinv_ref, b_ref, wc_ref, table_ref, z01_ref):
        dinv = dinv_ref[...]
        tsum = p_ref[0, :n, :] + p_ref[1, :n, :]
        h = jax.nn.relu(y01_ref[:, :dh] - dinv * tsum + b_ref[...])
        z = jnp.dot(h, wc_ref[...], preferred_element_type=jnp.float32)
        z01_ref[...] = z[:, :2 * dh]
        table_ref[...] = jnp.pad(dinv * z[:, 2 * dh:], ((0, n_pad - n), (0, 0)))
    return body


def _make_stage_final(n, dh):
    def body(z01_ref, p_ref, dinv_ref, b_ref, fw_ref, fb_ref, o_ref):
        dinv = dinv_ref[...]
        vsum = p_ref[0, :n, :] + p_ref[1, :n, :]
        h = jax.nn.relu(z01_ref[:, :dh] - dinv * vsum + b_ref[...])
        o_ref[...] = (
            jnp.dot(h, fw_ref[...], preferred_element_type=jnp.float32)
            + fb_ref[...]
        )
    return body


def kernel(x, edge_index, W1, b1, W2, b2, fc_w, fc_b):
    n, d_in = x.shape
    dh = W1.shape[2]
    e = edge_index.shape[1]

    epb = NW * 8 * LANE
    e_pad = -(-e // epb) * epb
    rows = e_pad // LANE
    rpt = rows // NW
    ei = jnp.pad(edge_index, ((0, 0), (0, e_pad - e)))
    ei = ei.reshape(2, rows, LANE)

    n_pad = -(-(n + 1) // (NS * 8)) * (NS * 8)
    rps = n_pad // NS

    idx3 = pl.pallas_call(
        _edge_prep(n),
        out_shape=jax.ShapeDtypeStruct((3, rows, LANE), jnp.int32),
    )(ei)
    src_g = idx3[0]
    dst_r = idx3[1]
    src_r = idx3[2]

    zeros16 = jnp.zeros((rps, 16), jnp.float32)
    ones16 = jnp.ones((LANE, 16), jnp.float32)
    zeros_dh = jnp.zeros((rps, dh), jnp.float32)

    deg_kernel = _make_deg_kernel(n_pad, rpt, rps)
    degp = deg_kernel(src_r, ones16, zeros16)

    prop_kernel = _make_prop_kernel(n_pad, rpt, rps, dh)

    def run_prop(tpad):
        return prop_kernel(tpad, src_g, dst_r, zeros_dh)

    table_shape = jax.ShapeDtypeStruct((n_pad, dh), jnp.float32)

    wc1 = jnp.concatenate([W1[0] - W1[2], W1[1], W1[2]], axis=1)
    tbl1, y01, dinv = pl.pallas_call(
        _make_stage_first(n, n_pad, dh),
        out_shape=(
            table_shape,
            jax.ShapeDtypeStruct((n, 2 * dh), jnp.float32),
            jax.ShapeDtypeStruct((n, 1), jnp.float32),
        ),
    )(x, wc1, degp)

    sp = run_prop(tbl1)
    tbl2 = pl.pallas_call(
        _make_stage_mid(n, n_pad, dh),
        out_shape=table_shape,
    )(y01, sp, dinv)

    tp = run_prop(tbl2)

    wc2 = jnp.concatenate([W2[0] - W2[2], W2[1], W2[2]], axis=1)
    tbl3, z01 = pl.pallas_call(
        _make_stage_layer(n, n_pad, dh),
        out_shape=(
            table_shape,
            jax.ShapeDtypeStruct((n, 2 * dh), jnp.float32),
        ),
    )(y01, tp, dinv, b1.reshape(1, dh), wc2)

    up = run_prop(tbl3)
    tbl4 = pl.pallas_call(
        _make_stage_mid(n, n_pad, dh),
        out_shape=table_shape,
    )(z01, up, dinv)

    vp = run_prop(tbl4)

    out = pl.pallas_call(
        _make_stage_final(n, dh),
        out_shape=jax.ShapeDtypeStruct((n, 1), jnp.float32),
    )(z01, vp, dinv, b2.reshape(1, dh), fc_w, fc_b.reshape(1, 1))
    return out
